# baseline (speedup 1.0000x reference)
import jax
import jax.numpy as jnp
from jax.experimental import pallas as pl
from jax.experimental.pallas import tpu as pltpu


def _fused_kernel(lbl_ref, x_ref, w1_ref, w2_ref, feat_ref, lab_ref):
    c, h, w = x_ref.shape[1:]
    xf = x_ref[0].reshape(c, h * w).astype(jnp.bfloat16)
    hid = jnp.dot(w1_ref[...], xf, preferred_element_type=jnp.float32)
    hb = jnp.maximum(hid, 0.0).astype(jnp.bfloat16)
    out = jnp.dot(w2_ref[...], hb, preferred_element_type=jnp.float32)
    feat_ref[0] = out.reshape(feat_ref.shape[1], h, w)
    b = pl.program_id(0)
    lab_ref[...] = jnp.full(lab_ref.shape, lbl_ref[b], dtype=jnp.int32)


def kernel(x, w1, w2, need_backprop):
    B, C, H, W = x.shape
    hidden = w1.shape[0]
    out_c = w2.shape[0]

    lbl = need_backprop.astype(jnp.float32).astype(jnp.int32)
    w1b = w1.astype(jnp.bfloat16)
    w2b = w2.astype(jnp.bfloat16)

    feat, label = pl.pallas_call(
        _fused_kernel,
        out_shape=(
            jax.ShapeDtypeStruct((B, out_c, H, W), x.dtype),
            jax.ShapeDtypeStruct((B, H, W), jnp.int32),
        ),
        grid_spec=pltpu.PrefetchScalarGridSpec(
            num_scalar_prefetch=1,
            grid=(B,),
            in_specs=[
                pl.BlockSpec((1, C, H, W), lambda b, lbl: (b, 0, 0, 0)),
                pl.BlockSpec((hidden, C), lambda b, lbl: (0, 0)),
                pl.BlockSpec((out_c, hidden), lambda b, lbl: (0, 0)),
            ],
            out_specs=(
                pl.BlockSpec((1, out_c, H, W), lambda b, lbl: (b, 0, 0, 0)),
                pl.BlockSpec((1, H, W), lambda b, lbl: (b, 0, 0)),
            ),
        ),
        compiler_params=pltpu.CompilerParams(
            dimension_semantics=("parallel",)),
    )(lbl, x, w1b, w2b)

    return feat, label
